# baseline (speedup 1.0000x reference)
import jax
import jax.numpy as jnp
from jax import lax
from jax.experimental import pallas as pl
from jax.experimental.pallas import tpu as pltpu
from jax.experimental.pallas import tpu_sc as plsc

N = 10000
E = 320000
D = 128

NC = 2
NS = 16
NW = NC * NS
EPW = E // NW
K = 100
NCH = EPW // K
RPT = N // NS
ZCH = 100
NZ = RPT // ZCH

L = 16


def _hist_body(ei_hbm, out_hbm, dst_v, hist_v):
    c = lax.axis_index("c")
    s = lax.axis_index("s")
    w = c * NS + s
    pltpu.sync_copy(ei_hbm.at[1].at[pl.ds(w * EPW, EPW)], dst_v)

    def zero(i, _):
        hist_v[pl.ds(i * L, L)] = jnp.zeros((L,), jnp.float32)
        return 0

    lax.fori_loop(0, N // L, zero, 0)

    ones = jnp.ones((L,), jnp.float32)

    def body(i, _):
        idx = dst_v[pl.ds(i * L, L)]
        plsc.addupdate_scatter(hist_v, [idx], ones)
        return 0

    lax.fori_loop(0, EPW // L, body, 0, unroll=4)
    pltpu.sync_copy(hist_v, out_hbm.at[w])


def _agg_body(g_hbm, srcr_hbm, dstr_hbm, out_hbm, src_v, dst_v, rows_v,
              acc_sh, sem0, sem1):
    c = lax.axis_index("c")
    s = lax.axis_index("s")
    w = c * NS + s
    pltpu.sync_copy(srcr_hbm.at[w], src_v)
    pltpu.sync_copy(dstr_hbm.at[w], dst_v)

    sems = (sem0, sem1)
    pltpu.async_copy(g_hbm.at[src_v.at[0]], rows_v.at[0], sems[0])

    def zrow(i, _):
        for j in range(D // L):
            rows_v[1, i, pl.ds(j * L, L)] = jnp.zeros((L,), jnp.float32)
        return 0

    lax.fori_loop(0, ZCH, zrow, 0)
    for j in range(NZ):
        pltpu.sync_copy(rows_v.at[1], acc_sh.at[pl.ds(s * RPT + j * ZCH, ZCH)])
    tail = RPT - NZ * ZCH
    pltpu.sync_copy(rows_v.at[1].at[pl.ds(0, tail)],
                    acc_sh.at[pl.ds(s * RPT + NZ * ZCH, tail)])
    pltpu.async_copy(g_hbm.at[src_v.at[1]], rows_v.at[1], sems[1])
    plsc.subcore_barrier()


    def body(t, _):
        j = t * 2
        for b in range(2):
            pltpu.make_async_copy(
                g_hbm.at[src_v.at[j + b]], rows_v.at[b], sems[b]).wait()
            pltpu.sync_copy(rows_v.at[b], acc_sh.at[dst_v.at[j + b]],
                            add=True)

            @pl.when(j + b + 2 < NCH)
            def _():
                pltpu.async_copy(g_hbm.at[src_v.at[j + b + 2]],
                                 rows_v.at[b], sems[b])
        return 0

    lax.fori_loop(0, NCH // 2, body, 0)
    plsc.subcore_barrier()

    sl = pl.ds(s * RPT, RPT)
    pltpu.sync_copy(acc_sh.at[sl], out_hbm.at[c].at[sl])


_sc_mesh = plsc.VectorSubcoreMesh(
    core_axis_name="c", subcore_axis_name="s", num_cores=NC, num_subcores=NS)

_sc_params = pltpu.CompilerParams(
    needs_layout_passes=False, use_tc_tiling_on_sc=False)

_hist = pl.kernel(
    _hist_body,
    out_type=jax.ShapeDtypeStruct((NW, N), jnp.float32),
    mesh=_sc_mesh,
    compiler_params=_sc_params,
    scratch_types=[
        pltpu.VMEM((EPW,), jnp.int32),
        pltpu.VMEM((N,), jnp.float32),
    ],
)

_agg = pl.kernel(
    _agg_body,
    out_type=jax.ShapeDtypeStruct((NC, N, D), jnp.float32),
    mesh=_sc_mesh,
    compiler_params=_sc_params,
    scratch_types=[
        pltpu.VMEM((NCH, K), jnp.int32),
        pltpu.VMEM((NCH, K), jnp.int32),
        pltpu.VMEM((2, K, D), jnp.float32),
        pltpu.VMEM_SHARED((N, D), jnp.float32),
        pltpu.SemaphoreType.DMA,
        pltpu.SemaphoreType.DMA,
    ],
)


R = 2000


def _dense1_body(x_ref, degp_ref, gam_ref, bet_ref, wc_ref, wl_ref, bl_ref,
                 g_ref, hl_ref):
    x = x_ref[...]
    mu = jnp.mean(x, axis=1, keepdims=True)
    xc = x - mu
    var = jnp.mean(xc * xc, axis=1, keepdims=True)
    h = (xc * lax.rsqrt(var + 1e-5) * gam_ref[...][None, :]
         + bet_ref[...][None, :])
    deg = jnp.sum(degp_ref[...], axis=1) + 1.0
    dinv = lax.rsqrt(deg)
    hb = h.astype(jnp.bfloat16)
    hc = jnp.dot(hb, wc_ref[...].astype(jnp.bfloat16),
                 preferred_element_type=jnp.float32)
    g_ref[...] = hc * dinv[:, None]
    hl_ref[...] = (
        jnp.dot(hb, wl_ref[...].astype(jnp.bfloat16),
                preferred_element_type=jnp.float32)
        + bl_ref[...][None, :])


def _dense2_body(acc_ref, g_ref, hl_ref, degp_ref, bc_ref, wp_ref, bp_ref,
                 out_ref):
    deg = jnp.sum(degp_ref[...], axis=1) + 1.0
    dinv = lax.rsqrt(deg)
    a = acc_ref[0] + acc_ref[1] + g_ref[...]
    conv = a * dinv[:, None] + bc_ref[...][None, :]
    z = jnp.maximum(conv + hl_ref[...], 0.0)
    out_ref[...] = (
        jnp.dot(z.astype(jnp.bfloat16), wp_ref[...].astype(jnp.bfloat16),
                preferred_element_type=jnp.float32)
        + bp_ref[...][None, :])


_dense1 = pl.pallas_call(
    _dense1_body,
    grid=(N // R,),
    in_specs=[
        pl.BlockSpec((R, D), lambda i: (i, 0)),
        pl.BlockSpec((R, NW), lambda i: (i, 0)),
        pl.BlockSpec((D,), lambda i: (0,)),
        pl.BlockSpec((D,), lambda i: (0,)),
        pl.BlockSpec((D, D), lambda i: (0, 0)),
        pl.BlockSpec((D, D), lambda i: (0, 0)),
        pl.BlockSpec((D,), lambda i: (0,)),
    ],
    out_specs=[
        pl.BlockSpec((R, D), lambda i: (i, 0)),
        pl.BlockSpec((R, D), lambda i: (i, 0)),
    ],
    out_shape=[
        jax.ShapeDtypeStruct((N, D), jnp.float32),
        jax.ShapeDtypeStruct((N, D), jnp.float32),
    ],
)

_dense2 = pl.pallas_call(
    _dense2_body,
    grid=(N // R,),
    in_specs=[
        pl.BlockSpec((NC, R, D), lambda i: (0, i, 0)),
        pl.BlockSpec((R, D), lambda i: (i, 0)),
        pl.BlockSpec((R, D), lambda i: (i, 0)),
        pl.BlockSpec((R, NW), lambda i: (i, 0)),
        pl.BlockSpec((D,), lambda i: (0,)),
        pl.BlockSpec((D, D), lambda i: (0, 0)),
        pl.BlockSpec((D,), lambda i: (0,)),
    ],
    out_specs=pl.BlockSpec((R, D), lambda i: (i, 0)),
    out_shape=jax.ShapeDtypeStruct((N, D), jnp.float32),
)


@jax.jit
def kernel(x, edge_index, ln_gamma, ln_beta, W_conv, b_conv, W_lin, b_lin,
           W_pred, b_pred):
    srcr = edge_index[0].reshape(NW, NCH, K)
    dstr = edge_index[1].reshape(NW, NCH, K)
    degp = _hist(edge_index).T
    g, hl = _dense1(x, degp, ln_gamma, ln_beta, W_conv, W_lin, b_lin)
    acc = _agg(g, srcr, dstr)
    return _dense2(acc, g, hl, degp, b_conv, W_pred, b_pred)


# speedup vs baseline: 1.0335x; 1.0335x over previous
import jax
import jax.numpy as jnp
from jax import lax
from jax.experimental import pallas as pl
from jax.experimental.pallas import tpu as pltpu
from jax.experimental.pallas import tpu_sc as plsc

N = 10000
E = 320000
D = 128

NC = 2
NS = 16
NW = NC * NS
EPW = E // NW
K = 100
NCH = EPW // K
RPT = N // NS
ZCH = 100
NZ = RPT // ZCH

L = 16


def _hist_body(ei_hbm, out_hbm, dst_v, hist_v):
    c = lax.axis_index("c")
    s = lax.axis_index("s")
    w = c * NS + s
    pltpu.sync_copy(ei_hbm.at[1].at[pl.ds(w * EPW, EPW)], dst_v)

    def zero(i, _):
        hist_v[pl.ds(i * L, L)] = jnp.zeros((L,), jnp.float32)
        return 0

    lax.fori_loop(0, N // L, zero, 0)

    ones = jnp.ones((L,), jnp.float32)

    def body(i, _):
        idx = dst_v[pl.ds(i * L, L)]
        plsc.addupdate_scatter(hist_v, [idx], ones)
        return 0

    lax.fori_loop(0, EPW // L, body, 0, unroll=4)
    pltpu.sync_copy(hist_v, out_hbm.at[w])


def _agg_body(g_hbm, eir_hbm, out_hbm, src_v, dst_v, rows_v,
              acc_sh, sem0, sem1):
    c = lax.axis_index("c")
    s = lax.axis_index("s")
    w = c * NS + s
    pltpu.sync_copy(eir_hbm.at[0].at[pl.ds(w * NCH, NCH)], src_v)
    pltpu.sync_copy(eir_hbm.at[1].at[pl.ds(w * NCH, NCH)], dst_v)

    sems = (sem0, sem1)
    pltpu.async_copy(g_hbm.at[src_v.at[0]], rows_v.at[0], sems[0])

    def zrow(i, _):
        for j in range(D // L):
            rows_v[1, i, pl.ds(j * L, L)] = jnp.zeros((L,), jnp.float32)
        return 0

    lax.fori_loop(0, ZCH, zrow, 0)
    for j in range(NZ):
        pltpu.sync_copy(rows_v.at[1], acc_sh.at[pl.ds(s * RPT + j * ZCH, ZCH)])
    tail = RPT - NZ * ZCH
    pltpu.sync_copy(rows_v.at[1].at[pl.ds(0, tail)],
                    acc_sh.at[pl.ds(s * RPT + NZ * ZCH, tail)])
    pltpu.async_copy(g_hbm.at[src_v.at[1]], rows_v.at[1], sems[1])
    plsc.subcore_barrier()


    def body(t, _):
        j = t * 2
        for b in range(2):
            pltpu.make_async_copy(
                g_hbm.at[src_v.at[j + b]], rows_v.at[b], sems[b]).wait()
            pltpu.sync_copy(rows_v.at[b], acc_sh.at[dst_v.at[j + b]],
                            add=True)

            @pl.when(j + b + 2 < NCH)
            def _():
                pltpu.async_copy(g_hbm.at[src_v.at[j + b + 2]],
                                 rows_v.at[b], sems[b])
        return 0

    lax.fori_loop(0, NCH // 2, body, 0)
    plsc.subcore_barrier()

    sl = pl.ds(s * RPT, RPT)
    pltpu.sync_copy(acc_sh.at[sl], out_hbm.at[c].at[sl])


_sc_mesh = plsc.VectorSubcoreMesh(
    core_axis_name="c", subcore_axis_name="s", num_cores=NC, num_subcores=NS)

_sc_params = pltpu.CompilerParams(
    needs_layout_passes=False, use_tc_tiling_on_sc=False)

_hist = pl.kernel(
    _hist_body,
    out_type=jax.ShapeDtypeStruct((NW, N), jnp.float32),
    mesh=_sc_mesh,
    compiler_params=_sc_params,
    scratch_types=[
        pltpu.VMEM((EPW,), jnp.int32),
        pltpu.VMEM((N,), jnp.float32),
    ],
)

_agg = pl.kernel(
    _agg_body,
    out_type=jax.ShapeDtypeStruct((NC, N, D), jnp.float32),
    mesh=_sc_mesh,
    compiler_params=_sc_params,
    scratch_types=[
        pltpu.VMEM((NCH, K), jnp.int32),
        pltpu.VMEM((NCH, K), jnp.int32),
        pltpu.VMEM((2, K, D), jnp.float32),
        pltpu.VMEM_SHARED((N, D), jnp.float32),
        pltpu.SemaphoreType.DMA,
        pltpu.SemaphoreType.DMA,
    ],
)


R = 2000


def _dense1_body(x_ref, degp_ref, gam_ref, bet_ref, wc_ref, wl_ref, bl_ref,
                 g_ref, hl_ref):
    x = x_ref[...]
    mu = jnp.mean(x, axis=1, keepdims=True)
    xc = x - mu
    var = jnp.mean(xc * xc, axis=1, keepdims=True)
    h = (xc * lax.rsqrt(var + 1e-5) * gam_ref[...][None, :]
         + bet_ref[...][None, :])
    deg = jnp.sum(degp_ref[...], axis=1) + 1.0
    dinv = lax.rsqrt(deg)
    hc = jnp.dot(h, wc_ref[...], preferred_element_type=jnp.float32)
    g_ref[...] = hc * dinv[:, None]
    hl_ref[...] = (
        jnp.dot(h, wl_ref[...], preferred_element_type=jnp.float32)
        + bl_ref[...][None, :])


def _dense2_body(acc_ref, g_ref, hl_ref, degp_ref, bc_ref, wp_ref, bp_ref,
                 out_ref):
    deg = jnp.sum(degp_ref[...], axis=1) + 1.0
    dinv = lax.rsqrt(deg)
    a = acc_ref[0] + acc_ref[1] + g_ref[...]
    conv = a * dinv[:, None] + bc_ref[...][None, :]
    z = jnp.maximum(conv + hl_ref[...], 0.0)
    out_ref[...] = (
        jnp.dot(z, wp_ref[...], preferred_element_type=jnp.float32)
        + bp_ref[...][None, :])


_dense1 = pl.pallas_call(
    _dense1_body,
    grid=(N // R,),
    in_specs=[
        pl.BlockSpec((R, D), lambda i: (i, 0)),
        pl.BlockSpec((R, NW), lambda i: (i, 0)),
        pl.BlockSpec((D,), lambda i: (0,)),
        pl.BlockSpec((D,), lambda i: (0,)),
        pl.BlockSpec((D, D), lambda i: (0, 0)),
        pl.BlockSpec((D, D), lambda i: (0, 0)),
        pl.BlockSpec((D,), lambda i: (0,)),
    ],
    out_specs=[
        pl.BlockSpec((R, D), lambda i: (i, 0)),
        pl.BlockSpec((R, D), lambda i: (i, 0)),
    ],
    out_shape=[
        jax.ShapeDtypeStruct((N, D), jnp.float32),
        jax.ShapeDtypeStruct((N, D), jnp.float32),
    ],
)

_dense2 = pl.pallas_call(
    _dense2_body,
    grid=(N // R,),
    in_specs=[
        pl.BlockSpec((NC, R, D), lambda i: (0, i, 0)),
        pl.BlockSpec((R, D), lambda i: (i, 0)),
        pl.BlockSpec((R, D), lambda i: (i, 0)),
        pl.BlockSpec((R, NW), lambda i: (i, 0)),
        pl.BlockSpec((D,), lambda i: (0,)),
        pl.BlockSpec((D, D), lambda i: (0, 0)),
        pl.BlockSpec((D,), lambda i: (0,)),
    ],
    out_specs=pl.BlockSpec((R, D), lambda i: (i, 0)),
    out_shape=jax.ShapeDtypeStruct((N, D), jnp.float32),
)


@jax.jit
def kernel(x, edge_index, ln_gamma, ln_beta, W_conv, b_conv, W_lin, b_lin,
           W_pred, b_pred):
    eir = edge_index.reshape(2, E // K, K)
    degp = _hist(edge_index).T
    g, hl = _dense1(x, degp, ln_gamma, ln_beta, W_conv, W_lin, b_lin)
    acc = _agg(g, eir)
    return _dense2(acc, g, hl, degp, b_conv, W_pred, b_pred)
